# baseline (speedup 1.0000x reference)
import functools

import jax
import jax.numpy as jnp
from jax import lax
from jax.experimental import pallas as pl
from jax.experimental.pallas import tpu as pltpu
from jax.experimental.pallas import tpu_sc as plsc

_K = 1024
_C = 64
_N = 16384
_COMMIT = 0.25

_BLK = 2048
_NB = _N // _BLK
_R = 128

_NC = 2
_NS = 16
_NW = _NC * _NS
_BPW = _N // _NW
_CHUNK = 128
_NCHUNK = _BPW // _CHUNK

_CPAD = 128


def _dist_kernel(x_ref, w_ref, sw_ref, idx_ref, loss_ref, table_ref):
    flat = jnp.transpose(x_ref[0], (1, 0))
    sx = jnp.sum(flat ** 2, axis=1, keepdims=True)
    m = lax.dot_general(flat * 2.0, w_ref[:], (((1,), (1,)), ((), ())),
                        preferred_element_type=jnp.float32)
    kio = lax.broadcasted_iota(jnp.int32, (_R, 128), 1)
    kios = [kio + c * 128 for c in range(_K // 128)]
    total = jnp.float32(0.0)
    for r0 in range(0, _BLK, _R):
        sxr = jnp.broadcast_to(sx[r0:r0 + _R], (_R, 128))
        mv = sxr - m[r0:r0 + _R, 0:128] + sw_ref[:, 0:128]
        mi = kios[0]
        for c in range(1, _K // 128):
            d = sxr - m[r0:r0 + _R, c * 128:(c + 1) * 128] \
                + sw_ref[:, c * 128:(c + 1) * 128]
            lt = d < mv
            mv = jnp.where(lt, d, mv)
            mi = jnp.where(lt, kios[c], mi)
        dmin = jnp.min(mv, axis=1, keepdims=True)
        idx_ref[0, 0, r0:r0 + _R] = jnp.min(
            jnp.where(mv == dmin, mi, _K), axis=1)
        total += jnp.sum(dmin)

    @pl.when(pl.program_id(0) == 0)
    def _init():
        loss_ref[0, 0] = 0.0
        wb = w_ref[:].astype(jnp.bfloat16).astype(jnp.float32)
        table_ref[:, 0:_C] = wb
        table_ref[:, _C:_CPAD] = jnp.zeros((_K, _CPAD - _C), jnp.float32)

    loss_ref[0, 0] += total


def _argmin_loss(x3, emb_weight, sw):
    return pl.pallas_call(
        _dist_kernel,
        grid=(_NB,),
        in_specs=[
            pl.BlockSpec((1, _C, _BLK),
                         lambda i: (i // (4096 // _BLK), 0, i % (4096 // _BLK))),
            pl.BlockSpec((_K, _C), lambda i: (0, 0)),
            pl.BlockSpec((1, _K), lambda i: (0, 0)),
        ],
        out_specs=[
            pl.BlockSpec((1, 1, _BLK), lambda i: (i, 0, 0)),
            pl.BlockSpec(memory_space=pltpu.SMEM),
            pl.BlockSpec((_K, _CPAD), lambda i: (0, 0)),
        ],
        out_shape=[
            jax.ShapeDtypeStruct((_NB, 1, _BLK), jnp.int32),
            jax.ShapeDtypeStruct((1, 1), jnp.float32),
            jax.ShapeDtypeStruct((_K, _CPAD), jnp.float32),
        ],
    )(x3, emb_weight, sw)


_sc_mesh = plsc.VectorSubcoreMesh(core_axis_name="c", subcore_axis_name="s")


@functools.partial(
    pl.kernel,
    mesh=_sc_mesh,
    out_type=jax.ShapeDtypeStruct((_N, _CPAD), jnp.float32),
    scratch_types=[
        pltpu.VMEM((_BPW,), jnp.int32),
        pltpu.VMEM((_NCHUNK, _CHUNK, _CPAD), jnp.float32),
        pltpu.SemaphoreType.DMA,
        pltpu.SemaphoreType.DMA,
    ],
)
def _gather_rows(table_hbm, idx_hbm, out_hbm, idx_v, rows_v, gsem, wsem):
    wid = lax.axis_index("s") * _NC + lax.axis_index("c")
    base = wid * _BPW
    pltpu.sync_copy(idx_hbm.at[pl.ds(base, _BPW)], idx_v)
    copies = []
    for c in range(_NCHUNK):
        copies.append(pltpu.async_copy(
            table_hbm.at[idx_v.at[pl.ds(c * _CHUNK, _CHUNK)]],
            rows_v.at[c], gsem))
    writes = []
    for c in range(_NCHUNK):
        copies[c].wait()
        writes.append(pltpu.async_copy(
            rows_v.at[c], out_hbm.at[pl.ds(base + c * _CHUNK, _CHUNK)], wsem))
    for w in writes:
        w.wait()


def kernel(x, emb_weight):
    B, C, H, W, D = x.shape
    x3 = x.reshape(B, C, H * W * D)
    sw = jnp.sum(emb_weight ** 2, axis=1).reshape(1, _K)
    idx3, loss_sum, table = _argmin_loss(x3, emb_weight, sw)
    idx = idx3.reshape(_N)
    quant_flat = _gather_rows(table, idx)[:, :_C]
    quant = jnp.transpose(quant_flat.reshape(B, H, W, D, C), (0, 4, 1, 2, 3))
    quant_st = x + (quant - x)
    mean = loss_sum[0, 0] / jnp.float32(_N * _C)
    loss = mean + jnp.float32(_COMMIT) * mean
    return (quant_st, loss)


# speedup vs baseline: 1.1142x; 1.1142x over previous
import functools

import jax
import jax.numpy as jnp
from jax import lax
from jax.experimental import pallas as pl
from jax.experimental.pallas import tpu as pltpu
from jax.experimental.pallas import tpu_sc as plsc

_K = 1024
_C = 64
_N = 16384
_COMMIT = 0.25

_BLK = 2048
_NB = _N // _BLK
_R = 128

_NC = 2
_NS = 16
_NW = _NC * _NS
_BPW = _N // _NW
_CHUNK = 128
_NCHUNK = _BPW // _CHUNK

_CPAD = 128


def _dist_kernel(flat_ref, w_ref, sw_ref, idx_ref, loss_ref):
    flat = flat_ref[:]
    sx = jnp.sum(flat ** 2, axis=1, keepdims=True)
    m = lax.dot_general(flat * 2.0, w_ref[:], (((1,), (1,)), ((), ())),
                        preferred_element_type=jnp.float32)
    kio = lax.broadcasted_iota(jnp.int32, (_R, 128), 1)
    kios = [kio + c * 128 for c in range(_K // 128)]
    total = jnp.float32(0.0)
    for r0 in range(0, _BLK, _R):
        sxr = jnp.broadcast_to(sx[r0:r0 + _R], (_R, 128))
        mv = sxr - m[r0:r0 + _R, 0:128] + sw_ref[:, 0:128]
        mi = kios[0]
        for c in range(1, _K // 128):
            d = sxr - m[r0:r0 + _R, c * 128:(c + 1) * 128] \
                + sw_ref[:, c * 128:(c + 1) * 128]
            lt = d < mv
            mv = jnp.where(lt, d, mv)
            mi = jnp.where(lt, kios[c], mi)
        dmin = jnp.min(mv, axis=1, keepdims=True)
        idx_ref[0, 0, r0:r0 + _R] = jnp.min(
            jnp.where(mv == dmin, mi, _K), axis=1)
        total += jnp.sum(dmin)

    @pl.when(pl.program_id(0) == 0)
    def _init():
        loss_ref[0, 0] = 0.0

    loss_ref[0, 0] += total


_NH = _N // 2
_NBH = _NH // _BLK


def _argmin_loss(flat, emb_weight, sw, h):
    return pl.pallas_call(
        _dist_kernel,
        grid=(_NBH,),
        in_specs=[
            pl.BlockSpec((_BLK, _C), lambda i, _h=h: (_h * _NBH + i, 0)),
            pl.BlockSpec((_K, _C), lambda i: (0, 0)),
            pl.BlockSpec((1, _K), lambda i: (0, 0)),
        ],
        out_specs=[
            pl.BlockSpec((1, 1, _BLK), lambda i: (i, 0, 0)),
            pl.BlockSpec(memory_space=pltpu.SMEM),
        ],
        out_shape=[
            jax.ShapeDtypeStruct((_NBH, 1, _BLK), jnp.int32),
            jax.ShapeDtypeStruct((1, 1), jnp.float32),
        ],
    )(flat, emb_weight, sw)


_sc_mesh = plsc.VectorSubcoreMesh(core_axis_name="c", subcore_axis_name="s")


_BPWH = _NH // _NW
_NCHUNKH = _BPWH // _CHUNK


@functools.partial(
    pl.kernel,
    mesh=_sc_mesh,
    out_type=jax.ShapeDtypeStruct((_NH, _CPAD), jnp.float32),
    scratch_types=[
        pltpu.VMEM((_CHUNK,), jnp.int32),
        pltpu.VMEM((_CHUNK, _CPAD), jnp.float32),
        pltpu.SemaphoreType.DMA,
    ],
)
def _gather_rows(table_hbm, idx_hbm, out_hbm, idx_v, rows_v, sem):
    wid = lax.axis_index("s") * _NC + lax.axis_index("c")
    base = wid * _BPWH
    for c in range(_NCHUNKH):
        off = base + c * _CHUNK
        pltpu.sync_copy(idx_hbm.at[pl.ds(off, _CHUNK)], idx_v)
        pltpu.async_copy(table_hbm.at[idx_v], rows_v, sem).wait()
        pltpu.sync_copy(rows_v, out_hbm.at[pl.ds(off, _CHUNK)])


def kernel(x, emb_weight):
    B, C, H, W, D = x.shape
    flat = jnp.transpose(x, (0, 2, 3, 4, 1)).reshape(-1, C)
    sw = jnp.sum(emb_weight ** 2, axis=1).reshape(1, _K)
    table = emb_weight.astype(jnp.bfloat16).astype(jnp.float32)
    table = jnp.pad(table, ((0, 0), (0, _CPAD - _C)))
    halves = []
    loss_sums = []
    bh = B // 2
    for h in range(2):
        idx3, loss_sum = _argmin_loss(flat, emb_weight, sw, h)
        loss_sums.append(loss_sum[0, 0])
        quant_flat = _gather_rows(table, idx3.reshape(_NH))[:, :_C]
        quant = jnp.transpose(quant_flat.reshape(bh, H, W, D, C),
                              (0, 4, 1, 2, 3))
        xh = x[h * bh:(h + 1) * bh]
        halves.append(xh + (quant - xh))
    quant_st = jnp.concatenate(halves, axis=0)
    mean = (loss_sums[0] + loss_sums[1]) / jnp.float32(_N * _C)
    loss = mean + jnp.float32(_COMMIT) * mean
    return (quant_st, loss)


# speedup vs baseline: 1.1384x; 1.0217x over previous
import functools

import jax
import jax.numpy as jnp
from jax import lax
from jax.experimental import pallas as pl
from jax.experimental.pallas import tpu as pltpu
from jax.experimental.pallas import tpu_sc as plsc

_K = 1024
_C = 64
_N = 16384
_COMMIT = 0.25

_BLK = 4096
_NB = _N // _BLK
_R = 128

_NC = 2
_NS = 16
_NW = _NC * _NS
_BPW = _N // _NW
_CHUNK = 128
_NCHUNK = _BPW // _CHUNK

_CPAD = 128


def _dist_kernel(flat_ref, w_ref, sw_ref, idx_ref, loss_ref):
    flat = flat_ref[:]
    sx = jnp.sum(flat ** 2, axis=1, keepdims=True)
    m = lax.dot_general(flat * 2.0, w_ref[:], (((1,), (1,)), ((), ())),
                        preferred_element_type=jnp.float32)
    kio = lax.broadcasted_iota(jnp.int32, (_R, 128), 1)
    kios = [kio + c * 128 for c in range(_K // 128)]
    total = jnp.float32(0.0)
    for r0 in range(0, _BLK, _R):
        sxr = jnp.broadcast_to(sx[r0:r0 + _R], (_R, 128))
        mv = sxr - m[r0:r0 + _R, 0:128] + sw_ref[:, 0:128]
        mi = kios[0]
        for c in range(1, _K // 128):
            d = sxr - m[r0:r0 + _R, c * 128:(c + 1) * 128] \
                + sw_ref[:, c * 128:(c + 1) * 128]
            lt = d < mv
            mv = jnp.where(lt, d, mv)
            mi = jnp.where(lt, kios[c], mi)
        dmin = jnp.min(mv, axis=1, keepdims=True)
        idx_ref[0, 0, r0:r0 + _R] = jnp.min(
            jnp.where(mv == dmin, mi, _K), axis=1)
        total += jnp.sum(dmin)

    @pl.when(pl.program_id(0) == 0)
    def _init():
        loss_ref[0, 0] = 0.0

    loss_ref[0, 0] += total


def _argmin_loss(flat, emb_weight, sw):
    return pl.pallas_call(
        _dist_kernel,
        grid=(_NB,),
        in_specs=[
            pl.BlockSpec((_BLK, _C), lambda i: (i, 0)),
            pl.BlockSpec((_K, _C), lambda i: (0, 0)),
            pl.BlockSpec((1, _K), lambda i: (0, 0)),
        ],
        out_specs=[
            pl.BlockSpec((1, 1, _BLK), lambda i: (i, 0, 0)),
            pl.BlockSpec(memory_space=pltpu.SMEM),
        ],
        out_shape=[
            jax.ShapeDtypeStruct((_NB, 1, _BLK), jnp.int32),
            jax.ShapeDtypeStruct((1, 1), jnp.float32),
        ],
    )(flat, emb_weight, sw)


_sc_mesh = plsc.VectorSubcoreMesh(core_axis_name="c", subcore_axis_name="s")


@functools.partial(
    pl.kernel,
    mesh=_sc_mesh,
    out_type=jax.ShapeDtypeStruct((_N, _CPAD), jnp.float32),
    scratch_types=[
        pltpu.VMEM((_CHUNK,), jnp.int32),
        pltpu.VMEM((_CHUNK, _CPAD), jnp.float32),
        pltpu.SemaphoreType.DMA,
    ],
)
def _gather_rows(table_hbm, idx_hbm, out_hbm, idx_v, rows_v, sem):
    wid = lax.axis_index("s") * _NC + lax.axis_index("c")
    base = wid * _BPW
    for c in range(_NCHUNK):
        off = base + c * _CHUNK
        pltpu.sync_copy(idx_hbm.at[pl.ds(off, _CHUNK)], idx_v)
        pltpu.async_copy(table_hbm.at[idx_v], rows_v, sem).wait()
        pltpu.sync_copy(rows_v, out_hbm.at[pl.ds(off, _CHUNK)])


def kernel(x, emb_weight):
    B, C, H, W, D = x.shape
    flat = jnp.transpose(x, (0, 2, 3, 4, 1)).reshape(-1, C)
    sw = jnp.sum(emb_weight ** 2, axis=1).reshape(1, _K)
    idx3, loss_sum = _argmin_loss(flat, emb_weight, sw)
    idx = idx3.reshape(_N)
    table = emb_weight.astype(jnp.bfloat16).astype(jnp.float32)
    table = jnp.pad(table, ((0, 0), (0, _CPAD - _C)))
    quant_flat = _gather_rows(table, idx)[:, :_C]
    quant = jnp.transpose(quant_flat.reshape(B, H, W, D, C), (0, 4, 1, 2, 3))
    quant_st = x + (quant - x)
    mean = loss_sum[0, 0] / jnp.float32(_N * _C)
    loss = mean + jnp.float32(_COMMIT) * mean
    return (quant_st, loss)
